# Initial kernel scaffold; baseline (speedup 1.0000x reference)
import jax
import jax.numpy as jnp
from jax.experimental import pallas as pl


def kernel(idx, table):
    raise NotImplementedError("write your pallas kernel here")


# speedup vs baseline: 1.8429x; 1.8429x over previous
import functools

import jax
import jax.numpy as jnp
from jax import lax
from jax.experimental import pallas as pl
from jax.experimental.pallas import tpu as pltpu
from jax.experimental.pallas import tpu_sc as plsc

CORPUS = 1_000_000
D = 64
B = 16384 * 50
NW = 32
BPW = B // NW
CHUNK = 512
NCHUNK = BPW // CHUNK

_mesh = plsc.VectorSubcoreMesh(core_axis_name="c", subcore_axis_name="s")


@functools.partial(
    pl.kernel,
    out_type=jax.ShapeDtypeStruct((B, D), jnp.float32),
    mesh=_mesh,
    scratch_types=[
        pltpu.VMEM((BPW,), jnp.int32),
        pltpu.VMEM((CHUNK, D), jnp.float32),
        pltpu.SemaphoreType.DMA,
    ],
    compiler_params=pltpu.CompilerParams(use_tc_tiling_on_sc=False),
)
def _emb_lookup(idx_hbm, table_hbm, out_hbm, idx_v, rows_v, sem):
    wid = lax.axis_index("s") * 2 + lax.axis_index("c")
    base = wid * BPW
    pltpu.sync_copy(idx_hbm.at[pl.ds(base, BPW)], idx_v)

    def body(g, _):
        off = g * CHUNK
        pltpu.async_copy(
            table_hbm.at[idx_v.at[pl.ds(off, CHUNK)]], rows_v, sem
        ).wait()
        pltpu.sync_copy(rows_v, out_hbm.at[pl.ds(base + off, CHUNK)])
        return 0

    lax.fori_loop(0, NCHUNK, body, 0)


def kernel(idx, table):
    idx_flat = idx.reshape(-1).astype(jnp.int32)
    out = _emb_lookup(idx_flat, table)
    return out.reshape(idx.shape + (D,))


# speedup vs baseline: 1.8650x; 1.0120x over previous
import functools

import jax
import jax.numpy as jnp
from jax import lax
from jax.experimental import pallas as pl
from jax.experimental.pallas import tpu as pltpu
from jax.experimental.pallas import tpu_sc as plsc

CORPUS = 1_000_000
D = 64
B = 16384 * 50
NW = 32
BPW = B // NW
CHUNK = 640
NCHUNK = BPW // CHUNK
NPAIR = NCHUNK // 2

_mesh = plsc.VectorSubcoreMesh(core_axis_name="c", subcore_axis_name="s")


@functools.partial(
    pl.kernel,
    out_type=jax.ShapeDtypeStruct((B, D), jnp.float32),
    mesh=_mesh,
    scratch_types=[
        pltpu.VMEM((BPW,), jnp.int32),
        pltpu.VMEM((CHUNK, D), jnp.float32),
        pltpu.VMEM((CHUNK, D), jnp.float32),
        pltpu.SemaphoreType.DMA,
        pltpu.SemaphoreType.DMA,
        pltpu.SemaphoreType.DMA,
        pltpu.SemaphoreType.DMA,
    ],
    compiler_params=pltpu.CompilerParams(use_tc_tiling_on_sc=False),
)
def _emb_lookup(idx_hbm, table_hbm, out_hbm, idx_v, buf0, buf1, g0, g1, o0, o1):
    wid = lax.axis_index("s") * 2 + lax.axis_index("c")
    base = wid * BPW
    pltpu.sync_copy(idx_hbm.at[pl.ds(base, BPW)], idx_v)

    def gather(c, buf, sem):
        pltpu.async_copy(table_hbm.at[idx_v.at[pl.ds(c * CHUNK, CHUNK)]], buf, sem)

    def gather_wait(c, buf, sem):
        pltpu.make_async_copy(
            table_hbm.at[idx_v.at[pl.ds(c * CHUNK, CHUNK)]], buf, sem
        ).wait()

    def put(c, buf, sem):
        pltpu.async_copy(buf, out_hbm.at[pl.ds(base + c * CHUNK, CHUNK)], sem)

    def put_wait(c, buf, sem):
        pltpu.make_async_copy(
            buf, out_hbm.at[pl.ds(base + c * CHUNK, CHUNK)], sem
        ).wait()

    gather(0, buf0, g0)
    gather(1, buf1, g1)

    def body(p, _):
        a = 2 * p
        gather_wait(a, buf0, g0)
        put(a, buf0, o0)
        gather_wait(a + 1, buf1, g1)
        put(a + 1, buf1, o1)

        @pl.when(p < NPAIR - 1)
        def _():
            put_wait(a, buf0, o0)
            gather(a + 2, buf0, g0)
            put_wait(a + 1, buf1, o1)
            gather(a + 3, buf1, g1)

        return 0

    lax.fori_loop(0, NPAIR, body, 0)
    put_wait(NCHUNK - 2, buf0, o0)
    put_wait(NCHUNK - 1, buf1, o1)


def kernel(idx, table):
    idx_flat = idx.reshape(-1).astype(jnp.int32)
    out = _emb_lookup(idx_flat, table)
    return out.reshape(idx.shape + (D,))
